# baseline (speedup 1.0000x reference)
import jax
import jax.numpy as jnp
import numpy as np
from jax.experimental import pallas as pl
from jax.experimental.pallas import tpu as pltpu

_N = 50
_M = 100
_H = 512
_W = 512
_SEM = 133
_THING = 80
_STUFF = _SEM - _THING
_CH = _STUFF + _N
_WIN = 128
_P = 10
_G = (_CH + _P - 1) // _P
_NBUF = 4
_LOOK = 2
_EPS = 1000.0 * float(np.finfo(np.float32).eps)


def _resize_weights(out_pos, k, box_len):
    inv = jnp.float32(_M) / box_len
    kernel_scale = jnp.maximum(inv, 1.0)
    sample = (out_pos + 0.5) * inv - 0.5
    x = jnp.abs(sample - k) / kernel_scale
    w = jnp.maximum(0.0, 1.0 - x)
    total = jnp.sum(w, axis=0, keepdims=True)
    w = jnp.where(
        jnp.abs(total) > _EPS,
        w / jnp.where(total != 0.0, total, 1.0),
        0.0,
    )
    return w


def _row_start(y0):
    return jnp.minimum(y0 // 8, (_H - _WIN) // 8) * 8


def _body(sched_ref, classes_ref, boxes_ref, sem_hbm, mask_hbm, out_hbm,
          sem_buf, mask_buf, out_buf, isem, osem):
    i = pl.program_id(0)
    slot = jax.lax.rem(i, _NBUF)

    def _chan_info(t, p):
        ch = sched_ref[t * _P + p]
        inst = jnp.clip(ch - _STUFF, 0, _N - 1)
        c = classes_ref[inst]
        rs = _row_start(boxes_ref[inst, 1])
        return ch, inst, c, rs

    def _ins(t, do_start):
        st = jax.lax.rem(t, _NBUF)
        ch0 = sched_ref[t * _P]
        chL = sched_ref[t * _P + _P - 1]
        batch_stuff = (ch0 >= 0) & (chL == ch0 + _P - 1) & (chL < _STUFF)
        batch_cp = pltpu.make_async_copy(
            sem_hbm.at[0, pl.ds(jnp.clip(ch0, 0, _SEM - _P), _P)],
            out_buf.at[st], isem.at[st])

        @pl.when(batch_stuff)
        def _batched():
            if do_start:
                batch_cp.start()
            else:
                batch_cp.wait()

        @pl.when(jnp.logical_not(batch_stuff))
        def _single():
            for p in range(_P):
                ch, inst, c, rs = _chan_info(t, p)
                stuff_cp = pltpu.make_async_copy(
                    sem_hbm.at[0, jnp.clip(ch, 0, _SEM - 1)],
                    out_buf.at[st, p], isem.at[st])
                win_cp = pltpu.make_async_copy(
                    sem_hbm.at[0, c + _STUFF, pl.ds(rs, _WIN), :],
                    sem_buf.at[st, p], isem.at[st])
                mask_cp = pltpu.make_async_copy(
                    mask_hbm.at[inst, c], mask_buf.at[st, p], isem.at[st])

                @pl.when((ch >= 0) & (ch < _STUFF))
                def _stuff(stuff_cp=stuff_cp):
                    if do_start:
                        stuff_cp.start()
                    else:
                        stuff_cp.wait()

                @pl.when(ch >= _STUFF)
                def _thing(win_cp=win_cp, mask_cp=mask_cp):
                    if do_start:
                        win_cp.start()
                        mask_cp.start()
                    else:
                        win_cp.wait()
                        mask_cp.wait()

    def _outs(t, do_start):
        st = jax.lax.rem(t, _NBUF)
        ch0 = sched_ref[t * _P]
        chL = sched_ref[t * _P + _P - 1]
        batch = (ch0 >= 0) & (chL == ch0 + _P - 1)
        batch_cp = pltpu.make_async_copy(
            out_buf.at[st],
            out_hbm.at[0, pl.ds(jnp.clip(ch0, 0, _CH - _P), _P)],
            osem.at[st])

        @pl.when(batch)
        def _batched():
            if do_start:
                batch_cp.start()
            else:
                batch_cp.wait()

        @pl.when(jnp.logical_not(batch))
        def _single():
            for p in range(_P):
                ch = sched_ref[t * _P + p]
                cp = pltpu.make_async_copy(
                    out_buf.at[st, p],
                    out_hbm.at[0, jnp.clip(ch, 0, _CH - 1)], osem.at[st])

                @pl.when(ch >= 0)
                def _go(cp=cp):
                    if do_start:
                        cp.start()
                    else:
                        cp.wait()

    @pl.when(i == 0)
    def _prologue():
        for t in range(_LOOK):
            _ins(jnp.int32(t), True)

    @pl.when(i + _LOOK < _G)
    def _stage_ahead():
        @pl.when(i >= _NBUF - _LOOK)
        def _free_slot():
            _outs(i - (_NBUF - _LOOK), False)
        _ins(i + _LOOK, True)

    _ins(i, False)

    for p in range(_P):
        ch, inst, c, rs = _chan_info(i, p)

        @pl.when(ch >= _STUFF)
        def _compute(p=p, inst=inst, rs=rs):
            x0 = boxes_ref[inst, 0]
            y0 = boxes_ref[inst, 1]
            x1 = boxes_ref[inst, 2]
            y1 = boxes_ref[inst, 3]
            bw = (x1 - x0 + 1).astype(jnp.float32)
            bh = (y1 - y0 + 1).astype(jnp.float32)

            ky = jax.lax.broadcasted_iota(
                jnp.int32, (_M, _WIN), 0).astype(jnp.float32)
            jy = jax.lax.broadcasted_iota(jnp.int32, (_M, _WIN), 1) + rs
            wy = _resize_weights((jy - y0).astype(jnp.float32), ky, bh)
            wy = jnp.where((jy >= y0) & (jy <= y1), wy, 0.0)

            kx = jax.lax.broadcasted_iota(
                jnp.int32, (_M, _W), 0).astype(jnp.float32)
            jx = jax.lax.broadcasted_iota(jnp.int32, (_M, _W), 1)
            wx = _resize_weights((jx - x0).astype(jnp.float32), kx, bw)
            wx = jnp.where((jx >= x0) & (jx <= x1), wx, 0.0)

            f = mask_buf[slot, p]
            ty = jax.lax.dot_general(
                wy, f, (((0,), (0,)), ((), ())),
                precision=jax.lax.Precision.HIGHEST,
                preferred_element_type=jnp.float32,
            )
            res = jax.lax.dot_general(
                ty, wx, (((1,), (0,)), ((), ())),
                precision=jax.lax.Precision.HIGHEST,
                preferred_element_type=jnp.float32,
            )

            rows = jax.lax.broadcasted_iota(jnp.int32, (_WIN, _W), 0) + rs
            cols = jax.lax.broadcasted_iota(jnp.int32, (_WIN, _W), 1)
            inbox = ((rows >= y0) & (rows <= y1) &
                     (cols >= x0) & (cols <= x1))
            strip = res + jnp.where(inbox, sem_buf[slot, p], 0.0)

            out_buf[slot, p] = jnp.zeros((_H, _W), jnp.float32)
            out_buf[slot, p, pl.ds(rs, _WIN), :] = strip

    _outs(i, True)

    @pl.when(i == _G - 1)
    def _drain():
        for t in range(_G - _NBUF, _G):
            _outs(jnp.int32(t), False)


def kernel(mask_logits, sem_seg_logits, gt_boxes, gt_classes):
    classes = gt_classes.astype(jnp.int32)
    boxes = gt_boxes.astype(jnp.int32)

    sched_list = []
    for k in range(_N):
        sched_list.extend([k, _STUFF + k])
    sched_list.extend(range(_N, _STUFF))
    sched_list.extend([-1] * (_G * _P - len(sched_list)))
    sched = jnp.asarray(np.array(sched_list, dtype=np.int32))

    grid_spec = pltpu.PrefetchScalarGridSpec(
        num_scalar_prefetch=3,
        grid=(_G,),
        in_specs=[
            pl.BlockSpec(memory_space=pl.ANY),
            pl.BlockSpec(memory_space=pl.ANY),
        ],
        out_specs=pl.BlockSpec(memory_space=pl.ANY),
        scratch_shapes=[
            pltpu.VMEM((_NBUF, _P, _WIN, _W), jnp.float32),
            pltpu.VMEM((_NBUF, _P, _M, _M), jnp.float32),
            pltpu.VMEM((_NBUF, _P, _H, _W), jnp.float32),
            pltpu.SemaphoreType.DMA((_NBUF,)),
            pltpu.SemaphoreType.DMA((_NBUF,)),
        ],
    )
    out = pl.pallas_call(
        _body,
        grid_spec=grid_spec,
        out_shape=jax.ShapeDtypeStruct((1, _CH, _H, _W), jnp.float32),
    )(sched, classes, boxes, sem_seg_logits, mask_logits)
    return out


# speedup vs baseline: 1.0135x; 1.0135x over previous
import jax
import jax.numpy as jnp
import numpy as np
from jax.experimental import pallas as pl
from jax.experimental.pallas import tpu as pltpu

_N = 50
_M = 100
_H = 512
_W = 512
_SEM = 133
_THING = 80
_STUFF = _SEM - _THING
_CH = _STUFF + _N
_WIN = 128
_P = 5
_G = (_CH + _P - 1) // _P
_NBUF = 8
_LOOK = 4
_EPS = 1000.0 * float(np.finfo(np.float32).eps)


def _resize_weights(out_pos, k, box_len):
    inv = jnp.float32(_M) / box_len
    kernel_scale = jnp.maximum(inv, 1.0)
    sample = (out_pos + 0.5) * inv - 0.5
    x = jnp.abs(sample - k) / kernel_scale
    w = jnp.maximum(0.0, 1.0 - x)
    total = jnp.sum(w, axis=0, keepdims=True)
    w = jnp.where(
        jnp.abs(total) > _EPS,
        w / jnp.where(total != 0.0, total, 1.0),
        0.0,
    )
    return w


def _row_start(y0):
    return jnp.minimum(y0 // 8, (_H - _WIN) // 8) * 8


def _body(sched_ref, classes_ref, boxes_ref, sem_hbm, mask_hbm, out_hbm,
          sem_buf, mask_buf, out_buf, isem, osem):
    i = pl.program_id(0)
    slot = jax.lax.rem(i, _NBUF)

    def _chan_info(t, p):
        ch = sched_ref[t * _P + p]
        inst = jnp.clip(ch - _STUFF, 0, _N - 1)
        c = classes_ref[inst]
        rs = _row_start(boxes_ref[inst, 1])
        return ch, inst, c, rs

    def _ins(t, do_start):
        st = jax.lax.rem(t, _NBUF)
        ch0 = sched_ref[t * _P]
        chL = sched_ref[t * _P + _P - 1]
        batch_stuff = (ch0 >= 0) & (chL == ch0 + _P - 1) & (chL < _STUFF)
        batch_cp = pltpu.make_async_copy(
            sem_hbm.at[0, pl.ds(jnp.clip(ch0, 0, _SEM - _P), _P)],
            out_buf.at[st], isem.at[st])

        @pl.when(batch_stuff)
        def _batched():
            if do_start:
                batch_cp.start()
            else:
                batch_cp.wait()

        @pl.when(jnp.logical_not(batch_stuff))
        def _single():
            for p in range(_P):
                ch, inst, c, rs = _chan_info(t, p)
                stuff_cp = pltpu.make_async_copy(
                    sem_hbm.at[0, jnp.clip(ch, 0, _SEM - 1)],
                    out_buf.at[st, p], isem.at[st])
                win_cp = pltpu.make_async_copy(
                    sem_hbm.at[0, c + _STUFF, pl.ds(rs, _WIN), :],
                    sem_buf.at[st, p], isem.at[st])
                mask_cp = pltpu.make_async_copy(
                    mask_hbm.at[inst, c], mask_buf.at[st, p], isem.at[st])

                @pl.when((ch >= 0) & (ch < _STUFF))
                def _stuff(stuff_cp=stuff_cp):
                    if do_start:
                        stuff_cp.start()
                    else:
                        stuff_cp.wait()

                @pl.when(ch >= _STUFF)
                def _thing(win_cp=win_cp, mask_cp=mask_cp):
                    if do_start:
                        win_cp.start()
                        mask_cp.start()
                    else:
                        win_cp.wait()
                        mask_cp.wait()

    def _outs(t, do_start):
        st = jax.lax.rem(t, _NBUF)
        ch0 = sched_ref[t * _P]
        chL = sched_ref[t * _P + _P - 1]
        batch = (ch0 >= 0) & (chL == ch0 + _P - 1)
        batch_cp = pltpu.make_async_copy(
            out_buf.at[st],
            out_hbm.at[0, pl.ds(jnp.clip(ch0, 0, _CH - _P), _P)],
            osem.at[st])

        @pl.when(batch)
        def _batched():
            if do_start:
                batch_cp.start()
            else:
                batch_cp.wait()

        @pl.when(jnp.logical_not(batch))
        def _single():
            for p in range(_P):
                ch = sched_ref[t * _P + p]
                cp = pltpu.make_async_copy(
                    out_buf.at[st, p],
                    out_hbm.at[0, jnp.clip(ch, 0, _CH - 1)], osem.at[st])

                @pl.when(ch >= 0)
                def _go(cp=cp):
                    if do_start:
                        cp.start()
                    else:
                        cp.wait()

    @pl.when(i == 0)
    def _prologue():
        for t in range(_LOOK):
            _ins(jnp.int32(t), True)

    @pl.when(i + _LOOK < _G)
    def _stage_ahead():
        @pl.when(i >= _NBUF - _LOOK)
        def _free_slot():
            _outs(i - (_NBUF - _LOOK), False)
        _ins(i + _LOOK, True)

    _ins(i, False)

    for p in range(_P):
        ch, inst, c, rs = _chan_info(i, p)

        @pl.when(ch >= _STUFF)
        def _compute(p=p, inst=inst, rs=rs):
            x0 = boxes_ref[inst, 0]
            y0 = boxes_ref[inst, 1]
            x1 = boxes_ref[inst, 2]
            y1 = boxes_ref[inst, 3]
            bw = (x1 - x0 + 1).astype(jnp.float32)
            bh = (y1 - y0 + 1).astype(jnp.float32)

            ky = jax.lax.broadcasted_iota(
                jnp.int32, (_M, _WIN), 0).astype(jnp.float32)
            jy = jax.lax.broadcasted_iota(jnp.int32, (_M, _WIN), 1) + rs
            wy = _resize_weights((jy - y0).astype(jnp.float32), ky, bh)
            wy = jnp.where((jy >= y0) & (jy <= y1), wy, 0.0)

            kx = jax.lax.broadcasted_iota(
                jnp.int32, (_M, _W), 0).astype(jnp.float32)
            jx = jax.lax.broadcasted_iota(jnp.int32, (_M, _W), 1)
            wx = _resize_weights((jx - x0).astype(jnp.float32), kx, bw)
            wx = jnp.where((jx >= x0) & (jx <= x1), wx, 0.0)

            f = mask_buf[slot, p]
            ty = jax.lax.dot_general(
                wy, f, (((0,), (0,)), ((), ())),
                precision=jax.lax.Precision.HIGHEST,
                preferred_element_type=jnp.float32,
            )
            res = jax.lax.dot_general(
                ty, wx, (((1,), (0,)), ((), ())),
                precision=jax.lax.Precision.HIGHEST,
                preferred_element_type=jnp.float32,
            )

            rows = jax.lax.broadcasted_iota(jnp.int32, (_WIN, _W), 0) + rs
            cols = jax.lax.broadcasted_iota(jnp.int32, (_WIN, _W), 1)
            inbox = ((rows >= y0) & (rows <= y1) &
                     (cols >= x0) & (cols <= x1))
            strip = res + jnp.where(inbox, sem_buf[slot, p], 0.0)

            out_buf[slot, p] = jnp.zeros((_H, _W), jnp.float32)
            out_buf[slot, p, pl.ds(rs, _WIN), :] = strip

    _outs(i, True)

    @pl.when(i == _G - 1)
    def _drain():
        for t in range(_G - _NBUF, _G):
            _outs(jnp.int32(t), False)


def kernel(mask_logits, sem_seg_logits, gt_boxes, gt_classes):
    classes = gt_classes.astype(jnp.int32)
    boxes = gt_boxes.astype(jnp.int32)

    sched_list = []
    for k in range(_N):
        sched_list.extend([k, _STUFF + k])
    sched_list.extend(range(_N, _STUFF))
    sched_list.extend([-1] * (_G * _P - len(sched_list)))
    sched = jnp.asarray(np.array(sched_list, dtype=np.int32))

    grid_spec = pltpu.PrefetchScalarGridSpec(
        num_scalar_prefetch=3,
        grid=(_G,),
        in_specs=[
            pl.BlockSpec(memory_space=pl.ANY),
            pl.BlockSpec(memory_space=pl.ANY),
        ],
        out_specs=pl.BlockSpec(memory_space=pl.ANY),
        scratch_shapes=[
            pltpu.VMEM((_NBUF, _P, _WIN, _W), jnp.float32),
            pltpu.VMEM((_NBUF, _P, _M, _M), jnp.float32),
            pltpu.VMEM((_NBUF, _P, _H, _W), jnp.float32),
            pltpu.SemaphoreType.DMA((_NBUF,)),
            pltpu.SemaphoreType.DMA((_NBUF,)),
        ],
    )
    out = pl.pallas_call(
        _body,
        grid_spec=grid_spec,
        out_shape=jax.ShapeDtypeStruct((1, _CH, _H, _W), jnp.float32),
    )(sched, classes, boxes, sem_seg_logits, mask_logits)
    return out
